# baseline (speedup 1.0000x reference)
import functools

import jax
import jax.numpy as jnp
from jax import lax
from jax.experimental import pallas as pl
from jax.experimental.pallas import tpu as pltpu
from jax.experimental.pallas import tpu_sc as plsc

E = 320000
N = 10000
D = 128
NPAD = 10240
NC = 2
NS = 16
NW = NC * NS
CH = 128
NCH = E // CH
TPW = (NCH + NW - 1) // NW
ROWS_PER_TILE = NPAD // NS
INV_SQRT_D = 1.0 / (D ** 0.5)

_mesh = plsc.VectorSubcoreMesh(
    core_axis_name="c", subcore_axis_name="s", num_cores=NC, num_subcores=NS)

_sc_params = pltpu.CompilerParams(needs_layout_passes=False)

_f32 = jnp.float32


def _wid():
    return lax.axis_index("s") * NC + lax.axis_index("c")


def _zero_rows(buf, n_rows):
    zero16 = jnp.zeros((16,), _f32)

    @pl.loop(0, n_rows)
    def _(i):
        for j in range(D // 16):
            buf[i, pl.ds(j * 16, 16)] = zero16


def _zero_vec(buf, n):
    zero16 = jnp.zeros((16,), _f32)

    @pl.loop(0, n // 16)
    def _(i):
        buf[pl.ds(i * 16, 16)] = zero16


def _fill_vec(buf, n, val):
    v16 = jnp.full((16,), val, _f32)

    @pl.loop(0, n // 16)
    def _(i):
        buf[pl.ds(i * 16, 16)] = v16


def _scatter_max(lmax, r16, v16):
    def cond(m):
        return plsc.all_reduce_population_count(m)[0] > 0

    def body(m):
        cur = plsc.load_gather(lmax, [r16])
        need = jnp.logical_and(m, cur < v16)
        plsc.store_scatter(lmax, [r16], v16, mask=need)
        return need

    lax.while_loop(cond, body, jnp.ones((16,), jnp.bool_))


@functools.partial(
    pl.kernel,
    out_type=jax.ShapeDtypeStruct((NC, NPAD, D), _f32),
    mesh=_mesh,
    compiler_params=_sc_params,
    scratch_types=[
        pltpu.VMEM((CH, D), _f32),
        pltpu.VMEM((CH, D), _f32),
        pltpu.VMEM((CH,), jnp.int32),
        pltpu.VMEM((CH,), jnp.int32),
        pltpu.SemaphoreType.DMA,
        pltpu.SemaphoreType.DMA,
        pltpu.VMEM_SHARED((NPAD, D), _f32),
    ],
)
def _k1_segment_sum(msg_hbm, recv_hbm, s_out, mra, mrb, ria, rib, sma, smb,
                    spmem):
    cid = lax.axis_index("c")
    sid = lax.axis_index("s")
    wid = _wid()
    bufs = ((mra, ria, sma), (mrb, rib, smb))
    _zero_rows(mra, CH)
    for t in range(ROWS_PER_TILE // CH):
        pltpu.sync_copy(mra, spmem.at[pl.ds(sid * ROWS_PER_TILE + t * CH, CH)])
    plsc.subcore_barrier()

    pltpu.async_copy(recv_hbm.at[pl.ds(wid * CH, CH)], ria, sma)
    pltpu.async_copy(msg_hbm.at[pl.ds(wid * CH, CH)], mra, sma)

    @pl.loop(0, TPW)
    def _(t):
        ch = wid + NW * t
        for ph in range(2):
            m_, r_, s_ = bufs[ph]
            mo_, ro_, so_ = bufs[1 - ph]

            @pl.when(jnp.logical_and(t % 2 == ph, ch < NCH))
            def _():
                base = ch * CH
                pltpu.make_async_copy(
                    recv_hbm.at[pl.ds(base, CH)], r_, s_).wait()
                pltpu.make_async_copy(
                    msg_hbm.at[pl.ds(base, CH)], m_, s_).wait()
                chn = ch + NW

                @pl.when(chn < NCH)
                def _():
                    nbase = chn * CH
                    pltpu.async_copy(recv_hbm.at[pl.ds(nbase, CH)], ro_, so_)
                    pltpu.async_copy(msg_hbm.at[pl.ds(nbase, CH)], mo_, so_)

                pltpu.sync_copy(m_, spmem.at[r_], add=True)

    plsc.subcore_barrier()
    pltpu.sync_copy(
        spmem.at[pl.ds(sid * ROWS_PER_TILE, ROWS_PER_TILE)],
        s_out.at[cid, pl.ds(sid * ROWS_PER_TILE, ROWS_PER_TILE)])


def _k2_body(s0, s1, wq, bq, wk, bk, qk_out, c_out):
    s = s0[...] + s1[...]
    q = jnp.dot(s, wq[...], preferred_element_type=_f32) + bq[...][None, :]
    qk = lax.dot_general(q, wk[...], (((1,), (1,)), ((), ())),
                         preferred_element_type=_f32)
    qk_out[...] = qk * INV_SQRT_D
    c = jnp.dot(q, bk[...], preferred_element_type=_f32)
    c_out[...] = c * INV_SQRT_D


def _k2_tables(s0, s1, wq, bq, wk, bk):
    nblk = NPAD // 512
    return pl.pallas_call(
        _k2_body,
        grid=(nblk,),
        in_specs=[
            pl.BlockSpec((512, D), lambda i: (i, 0)),
            pl.BlockSpec((512, D), lambda i: (i, 0)),
            pl.BlockSpec((D, D), lambda i: (0, 0)),
            pl.BlockSpec((D,), lambda i: (0,)),
            pl.BlockSpec((D, D), lambda i: (0, 0)),
            pl.BlockSpec((D,), lambda i: (0,)),
        ],
        out_specs=[
            pl.BlockSpec((512, D), lambda i: (i, 0)),
            pl.BlockSpec((512,), lambda i: (i,)),
        ],
        out_shape=[
            jax.ShapeDtypeStruct((NPAD, D), _f32),
            jax.ShapeDtypeStruct((NPAD,), _f32),
        ],
    )(s0, s1, wq, bq, wk, bk)


@functools.partial(
    pl.kernel,
    out_type=[
        jax.ShapeDtypeStruct((E,), _f32),
        jax.ShapeDtypeStruct((NW * NPAD,), _f32),
    ],
    mesh=_mesh,
    compiler_params=_sc_params,
    scratch_types=[
        pltpu.VMEM((CH, D), _f32),
        pltpu.VMEM((CH, D), _f32),
        pltpu.VMEM((CH, D), _f32),
        pltpu.VMEM((CH, D), _f32),
        pltpu.VMEM((CH,), jnp.int32),
        pltpu.VMEM((CH,), jnp.int32),
        pltpu.VMEM((CH,), _f32),
        pltpu.VMEM((CH * 17,), _f32),
        pltpu.VMEM((NPAD,), _f32),
        pltpu.VMEM((NPAD,), _f32),
        pltpu.SemaphoreType.DMA,
        pltpu.SemaphoreType.DMA,
    ],
)
def _k3_scores(msg_hbm, recv_hbm, qk_hbm, c_hbm, sc_out, mx_out,
               mra, mrb, qka, qkb, ria, rib, sbuf, pbuf, ccopy, lmax,
               sma, smb):
    wid = _wid()
    bufs = ((mra, qka, ria, sma), (mrb, qkb, rib, smb))
    _fill_vec(lmax, NPAD, -3.0e38)
    pltpu.sync_copy(c_hbm, ccopy)

    pltpu.sync_copy(recv_hbm.at[pl.ds(wid * CH, CH)], ria)
    pltpu.async_copy(msg_hbm.at[pl.ds(wid * CH, CH)], mra, sma)
    pltpu.async_copy(qk_hbm.at[ria], qka, sma)

    iota = lax.broadcasted_iota(jnp.int32, (16,), 0)
    lane0 = iota == 0

    @pl.loop(0, TPW)
    def _(t):
        ch = wid + NW * t
        for ph in range(2):
            m_, q_, r_, s_ = bufs[ph]
            mo_, qo_, ro_, so_ = bufs[1 - ph]

            @pl.when(jnp.logical_and(t % 2 == ph, ch < NCH))
            def _():
                base = ch * CH
                pltpu.make_async_copy(
                    msg_hbm.at[pl.ds(base, CH)], m_, s_).wait()
                pltpu.make_async_copy(qk_hbm.at[r_], q_, s_).wait()
                chn = ch + NW

                @pl.when(chn < NCH)
                def _():
                    nbase = chn * CH
                    pltpu.sync_copy(recv_hbm.at[pl.ds(nbase, CH)], ro_)
                    pltpu.async_copy(msg_hbm.at[pl.ds(nbase, CH)], mo_, so_)
                    pltpu.async_copy(qk_hbm.at[ro_], qo_, so_)

                @pl.loop(0, CH, unroll=4)
                def _(e):
                    acc = jnp.zeros((16,), _f32)
                    for j in range(D // 16):
                        acc = acc + (m_[e, pl.ds(j * 16, 16)]
                                     * q_[e, pl.ds(j * 16, 16)])
                    pbuf[pl.ds(e * 17, 16)] = acc

                for g in range(CH // 16):
                    idx0 = (iota + g * 16) * 17
                    parts = []
                    for c0 in range(4):
                        a = plsc.load_gather(pbuf, [idx0 + c0])
                        for k in range(c0 + 4, 16, 4):
                            a = a + plsc.load_gather(pbuf, [idx0 + k])
                        parts.append(a)
                    sbuf[pl.ds(g * 16, 16)] = (
                        (parts[0] + parts[1]) + (parts[2] + parts[3]))

                for sub in range(CH // 16):
                    r16 = r_[pl.ds(sub * 16, 16)]
                    s16 = (sbuf[pl.ds(sub * 16, 16)]
                           + plsc.load_gather(ccopy, [r16]))
                    sbuf[pl.ds(sub * 16, 16)] = s16
                    _scatter_max(lmax, r16, s16)

                pltpu.sync_copy(sbuf, sc_out.at[pl.ds(base, CH)])

    pltpu.sync_copy(lmax, mx_out.at[pl.ds(wid * NPAD, NPAD)])


@functools.partial(
    pl.kernel,
    out_type=jax.ShapeDtypeStruct((NPAD,), _f32),
    mesh=_mesh,
    compiler_params=_sc_params,
    scratch_types=[
        pltpu.VMEM((NPAD,), _f32),
        pltpu.VMEM((NPAD // NW,), _f32),
    ],
)
def _k4b_merge_max(mx_hbm, m_out, mbuf, obuf):
    wid = _wid()
    span = NPAD // NW
    for p in range(NW):
        pltpu.sync_copy(mx_hbm.at[pl.ds(p * NPAD + wid * span, span)],
                        mbuf.at[pl.ds(p * span, span)])
    for i in range(span // 16):
        acc = mbuf[pl.ds(i * 16, 16)]
        for p in range(1, NW):
            acc = jnp.maximum(acc, mbuf[pl.ds(p * span + i * 16, 16)])
        obuf[pl.ds(i * 16, 16)] = acc
    pltpu.sync_copy(obuf, m_out.at[pl.ds(wid * span, span)])


@functools.partial(
    pl.kernel,
    out_type=jax.ShapeDtypeStruct((NC * NPAD,), _f32),
    mesh=_mesh,
    compiler_params=_sc_params,
    scratch_types=[
        pltpu.VMEM((NPAD,), _f32),
        pltpu.VMEM((CH,), _f32),
        pltpu.VMEM((CH,), _f32),
        pltpu.VMEM((CH,), jnp.int32),
        pltpu.VMEM((ROWS_PER_TILE,), _f32),
        pltpu.VMEM_SHARED((NPAD,), _f32),
    ],
)
def _k4c_z(sc_hbm, recv_hbm, m_hbm, z_out,
           mcopy, sbuf, exbuf, ridx, zb, spmem_z):
    cid = lax.axis_index("c")
    sid = lax.axis_index("s")
    wid = _wid()
    _zero_vec(zb, ROWS_PER_TILE)
    pltpu.sync_copy(zb, spmem_z.at[pl.ds(sid * ROWS_PER_TILE, ROWS_PER_TILE)])
    plsc.subcore_barrier()
    pltpu.sync_copy(m_hbm, mcopy)

    @pl.loop(0, TPW)
    def _(t):
        ch = wid + NW * t

        @pl.when(ch < NCH)
        def _():
            base = ch * CH
            pltpu.sync_copy(sc_hbm.at[pl.ds(base, CH)], sbuf)
            pltpu.sync_copy(recv_hbm.at[pl.ds(base, CH)], ridx)
            for sub in range(CH // 16):
                r16 = ridx[pl.ds(sub * 16, 16)]
                m16 = plsc.load_gather(mcopy, [r16])
                exbuf[pl.ds(sub * 16, 16)] = jnp.exp(
                    sbuf[pl.ds(sub * 16, 16)] - m16)
            pltpu.sync_copy(exbuf, spmem_z.at[ridx], add=True)

    plsc.subcore_barrier()
    pltpu.sync_copy(
        spmem_z.at[pl.ds(sid * ROWS_PER_TILE, ROWS_PER_TILE)],
        z_out.at[pl.ds(cid * NPAD + sid * ROWS_PER_TILE, ROWS_PER_TILE)])


@functools.partial(
    pl.kernel,
    out_type=[
        jax.ShapeDtypeStruct((NPAD,), _f32),
        jax.ShapeDtypeStruct((NPAD,), _f32),
    ],
    mesh=_mesh,
    compiler_params=_sc_params,
    scratch_types=[
        pltpu.VMEM((NC * (NPAD // NW),), _f32),
        pltpu.VMEM((NPAD // NW,), _f32),
        pltpu.VMEM((NPAD // NW,), _f32),
        pltpu.VMEM((NPAD // NW,), _f32),
    ],
)
def _k4d_r_t(z_hbm, m_hbm, v_out, t_out, zbuf, mbuf, rbuf, tbuf):
    wid = _wid()
    span = NPAD // NW
    for p in range(NC):
        pltpu.sync_copy(z_hbm.at[pl.ds(p * NPAD + wid * span, span)],
                        zbuf.at[pl.ds(p * span, span)])
    pltpu.sync_copy(m_hbm.at[pl.ds(wid * span, span)], mbuf)
    for i in range(span // 16):
        z = zbuf[pl.ds(i * 16, 16)] + zbuf[pl.ds(span + i * 16, 16)]
        r = 1.0 / (z + 1e-8)
        rbuf[pl.ds(i * 16, 16)] = r * jnp.exp(-mbuf[pl.ds(i * 16, 16)])
        tbuf[pl.ds(i * 16, 16)] = z * r
    pltpu.sync_copy(rbuf, v_out.at[pl.ds(wid * span, span)])
    pltpu.sync_copy(tbuf, t_out.at[pl.ds(wid * span, span)])


@functools.partial(
    pl.kernel,
    out_type=jax.ShapeDtypeStruct((NC, NPAD, D), _f32),
    mesh=_mesh,
    compiler_params=_sc_params,
    scratch_types=[
        pltpu.VMEM((CH, D), _f32),
        pltpu.VMEM((CH, D), _f32),
        pltpu.VMEM((CH,), jnp.int32),
        pltpu.VMEM((CH,), jnp.int32),
        pltpu.VMEM((CH,), _f32),
        pltpu.VMEM((CH,), _f32),
        pltpu.VMEM((CH * 17,), _f32),
        pltpu.VMEM((NPAD,), _f32),
        pltpu.SemaphoreType.DMA,
        pltpu.SemaphoreType.DMA,
        pltpu.VMEM_SHARED((NPAD, D), _f32),
    ],
)
def _k5_weighted_sum(msg_hbm, recv_hbm, sc_hbm, v_hbm, a_out,
                     mra, mrb, ria, rib, sca, scb, wexp, vcopy,
                     sma, smb, spmem):
    cid = lax.axis_index("c")
    sid = lax.axis_index("s")
    wid = _wid()
    bufs = ((mra, ria, sca, sma), (mrb, rib, scb, smb))
    _zero_rows(mra, CH)
    for t in range(ROWS_PER_TILE // CH):
        pltpu.sync_copy(mra, spmem.at[pl.ds(sid * ROWS_PER_TILE + t * CH, CH)])
    plsc.subcore_barrier()
    pltpu.sync_copy(v_hbm, vcopy)
    iota = lax.broadcasted_iota(jnp.int32, (16,), 0)

    pltpu.async_copy(recv_hbm.at[pl.ds(wid * CH, CH)], ria, sma)
    pltpu.async_copy(sc_hbm.at[pl.ds(wid * CH, CH)], sca, sma)
    pltpu.async_copy(msg_hbm.at[pl.ds(wid * CH, CH)], mra, sma)

    @pl.loop(0, TPW)
    def _(t):
        ch = wid + NW * t
        for ph in range(2):
            m_, r_, c_, s_ = bufs[ph]
            mo_, ro_, co_, so_ = bufs[1 - ph]

            @pl.when(jnp.logical_and(t % 2 == ph, ch < NCH))
            def _():
                base = ch * CH
                pltpu.make_async_copy(
                    recv_hbm.at[pl.ds(base, CH)], r_, s_).wait()
                pltpu.make_async_copy(
                    sc_hbm.at[pl.ds(base, CH)], c_, s_).wait()
                pltpu.make_async_copy(
                    msg_hbm.at[pl.ds(base, CH)], m_, s_).wait()
                chn = ch + NW

                @pl.when(chn < NCH)
                def _():
                    nbase = chn * CH
                    pltpu.async_copy(recv_hbm.at[pl.ds(nbase, CH)], ro_, so_)
                    pltpu.async_copy(sc_hbm.at[pl.ds(nbase, CH)], co_, so_)
                    pltpu.async_copy(msg_hbm.at[pl.ds(nbase, CH)], mo_, so_)

                for sub in range(CH // 16):
                    r16 = r_[pl.ds(sub * 16, 16)]
                    v16 = plsc.load_gather(vcopy, [r16])
                    ex16 = jnp.exp(c_[pl.ds(sub * 16, 16)])
                    w16 = ex16 * v16
                    idx0 = (iota + sub * 16) * 17
                    for k in range(16):
                        plsc.store_scatter(wexp, [idx0 + k], w16)

                @pl.loop(0, CH, unroll=4)
                def _(e):
                    wsplat = wexp[pl.ds(e * 17, 16)]
                    for j in range(D // 16):
                        m_[e, pl.ds(j * 16, 16)] = (
                            m_[e, pl.ds(j * 16, 16)] * wsplat)

                pltpu.sync_copy(m_, spmem.at[r_], add=True)

    plsc.subcore_barrier()
    pltpu.sync_copy(
        spmem.at[pl.ds(sid * ROWS_PER_TILE, ROWS_PER_TILE)],
        a_out.at[cid, pl.ds(sid * ROWS_PER_TILE, ROWS_PER_TILE)])


def _k6_body(a0, a1, t, wv, bv, wo, bo, out):
    a = a0[...] + a1[...]
    agg = (jnp.dot(a, wv[...], preferred_element_type=_f32)
           + t[...][:, None] * bv[...][None, :])
    out[...] = jnp.dot(agg, wo[...], preferred_element_type=_f32) + bo[...][None, :]


def _k6_project(a0, a1, t, wv, bv, wo, bo):
    nblk = NPAD // 512
    return pl.pallas_call(
        _k6_body,
        grid=(nblk,),
        in_specs=[
            pl.BlockSpec((512, D), lambda i: (i, 0)),
            pl.BlockSpec((512, D), lambda i: (i, 0)),
            pl.BlockSpec((512,), lambda i: (i,)),
            pl.BlockSpec((D, D), lambda i: (0, 0)),
            pl.BlockSpec((D,), lambda i: (0,)),
            pl.BlockSpec((D, D), lambda i: (0, 0)),
            pl.BlockSpec((D,), lambda i: (0,)),
        ],
        out_specs=pl.BlockSpec((512, D), lambda i: (i, 0)),
        out_shape=jax.ShapeDtypeStruct((NPAD, D), _f32),
    )(a0, a1, t, wv, bv, wo, bo)


def kernel(messages, receivers, num_segments, Wk, bk, Wv, bv, Wq, bq, Wo, bo):
    receivers = receivers.astype(jnp.int32)
    s_parts = _k1_segment_sum(messages, receivers)
    qk, c = _k2_tables(s_parts[0], s_parts[1], Wq, bq, Wk, bk)
    scores, mx = _k3_scores(messages, receivers, qk, c)
    m = _k4b_merge_max(mx)
    z = _k4c_z(scores, receivers, m)
    v, t = _k4d_r_t(z, m)
    a_parts = _k5_weighted_sum(messages, receivers, scores, v)
    out = _k6_project(a_parts[0], a_parts[1], t, Wv, bv, Wo, bo)
    return out[:N]


# speedup vs baseline: 1.0846x; 1.0846x over previous
import functools

import jax
import jax.numpy as jnp
from jax import lax
from jax.experimental import pallas as pl
from jax.experimental.pallas import tpu as pltpu
from jax.experimental.pallas import tpu_sc as plsc

E = 320000
N = 10000
D = 128
NPAD = 10240
NC = 2
NS = 16
NW = NC * NS
CH = 128
NCH = E // CH
TPW = (NCH + NW - 1) // NW
ROWS_PER_TILE = NPAD // NS
INV_SQRT_D = 1.0 / (D ** 0.5)

_mesh = plsc.VectorSubcoreMesh(
    core_axis_name="c", subcore_axis_name="s", num_cores=NC, num_subcores=NS)

_sc_params = pltpu.CompilerParams(needs_layout_passes=False)

_f32 = jnp.float32


def _wid():
    return lax.axis_index("s") * NC + lax.axis_index("c")


def _zero_rows(buf, n_rows):
    zero16 = jnp.zeros((16,), _f32)

    @pl.loop(0, n_rows)
    def _(i):
        for j in range(D // 16):
            buf[i, pl.ds(j * 16, 16)] = zero16


def _zero_vec(buf, n):
    zero16 = jnp.zeros((16,), _f32)

    @pl.loop(0, n // 16)
    def _(i):
        buf[pl.ds(i * 16, 16)] = zero16


def _fill_vec(buf, n, val):
    v16 = jnp.full((16,), val, _f32)

    @pl.loop(0, n // 16)
    def _(i):
        buf[pl.ds(i * 16, 16)] = v16


def _scatter_max(lmax, r16, v16):
    def cond(m):
        return plsc.all_reduce_population_count(m)[0] > 0

    def body(m):
        cur = plsc.load_gather(lmax, [r16])
        need = jnp.logical_and(m, cur < v16)
        plsc.store_scatter(lmax, [r16], v16, mask=need)
        return need

    lax.while_loop(cond, body, jnp.ones((16,), jnp.bool_))


@functools.partial(
    pl.kernel,
    out_type=jax.ShapeDtypeStruct((NC, NPAD, D), _f32),
    mesh=_mesh,
    compiler_params=_sc_params,
    scratch_types=[
        pltpu.VMEM((CH, D), _f32),
        pltpu.VMEM((CH, D), _f32),
        pltpu.VMEM((CH,), jnp.int32),
        pltpu.VMEM((CH,), jnp.int32),
        pltpu.SemaphoreType.DMA,
        pltpu.SemaphoreType.DMA,
        pltpu.SemaphoreType.DMA,
        pltpu.SemaphoreType.DMA,
        pltpu.VMEM_SHARED((NPAD, D), _f32),
    ],
)
def _k1_segment_sum(msg_hbm, recv_hbm, s_out, mra, mrb, ria, rib, sma, smb,
                    ssa, ssb, spmem):
    cid = lax.axis_index("c")
    sid = lax.axis_index("s")
    wid = _wid()
    bufs = ((mra, ria, sma, ssa), (mrb, rib, smb, ssb))
    _zero_rows(mra, CH)
    for t in range(ROWS_PER_TILE // CH):
        pltpu.sync_copy(mra, spmem.at[pl.ds(sid * ROWS_PER_TILE + t * CH, CH)])
    plsc.subcore_barrier()

    pltpu.async_copy(recv_hbm.at[pl.ds(wid * CH, CH)], ria, sma)
    pltpu.async_copy(msg_hbm.at[pl.ds(wid * CH, CH)], mra, sma)

    @pl.loop(0, TPW)
    def _(t):
        ch = wid + NW * t
        for ph in range(2):
            m_, r_, s_, ss_ = bufs[ph]
            mo_, ro_, so_, sso_ = bufs[1 - ph]

            @pl.when(jnp.logical_and(t % 2 == ph, ch < NCH))
            def _():
                base = ch * CH
                pltpu.make_async_copy(
                    recv_hbm.at[pl.ds(base, CH)], r_, s_).wait()
                pltpu.make_async_copy(
                    msg_hbm.at[pl.ds(base, CH)], m_, s_).wait()
                chn = ch + NW

                @pl.when(chn < NCH)
                def _():
                    @pl.when(t > 0)
                    def _():
                        pltpu.make_async_copy(
                            mo_, spmem.at[ro_], sso_).wait()

                    nbase = chn * CH
                    pltpu.async_copy(recv_hbm.at[pl.ds(nbase, CH)], ro_, so_)
                    pltpu.async_copy(msg_hbm.at[pl.ds(nbase, CH)], mo_, so_)

                pltpu.async_copy(m_, spmem.at[r_], ss_, add=True)

    pltpu.make_async_copy(mra, spmem.at[ria], ssa).wait()
    pltpu.make_async_copy(mrb, spmem.at[rib], ssb).wait()
    plsc.subcore_barrier()
    pltpu.sync_copy(
        spmem.at[pl.ds(sid * ROWS_PER_TILE, ROWS_PER_TILE)],
        s_out.at[cid, pl.ds(sid * ROWS_PER_TILE, ROWS_PER_TILE)])


def _k2_body(s0, s1, wq, bq, wk, bk, qk_out, c_out):
    s = s0[...] + s1[...]
    q = jnp.dot(s, wq[...], preferred_element_type=_f32) + bq[...][None, :]
    qk = lax.dot_general(q, wk[...], (((1,), (1,)), ((), ())),
                         preferred_element_type=_f32)
    qk_out[...] = qk * INV_SQRT_D
    c = jnp.dot(q, bk[...], preferred_element_type=_f32)
    c_out[...] = c * INV_SQRT_D


def _k2_tables(s0, s1, wq, bq, wk, bk):
    nblk = NPAD // 512
    return pl.pallas_call(
        _k2_body,
        grid=(nblk,),
        in_specs=[
            pl.BlockSpec((512, D), lambda i: (i, 0)),
            pl.BlockSpec((512, D), lambda i: (i, 0)),
            pl.BlockSpec((D, D), lambda i: (0, 0)),
            pl.BlockSpec((D,), lambda i: (0,)),
            pl.BlockSpec((D, D), lambda i: (0, 0)),
            pl.BlockSpec((D,), lambda i: (0,)),
        ],
        out_specs=[
            pl.BlockSpec((512, D), lambda i: (i, 0)),
            pl.BlockSpec((512,), lambda i: (i,)),
        ],
        out_shape=[
            jax.ShapeDtypeStruct((NPAD, D), _f32),
            jax.ShapeDtypeStruct((NPAD,), _f32),
        ],
    )(s0, s1, wq, bq, wk, bk)


@functools.partial(
    pl.kernel,
    out_type=[
        jax.ShapeDtypeStruct((E,), _f32),
        jax.ShapeDtypeStruct((NW * NPAD,), _f32),
    ],
    mesh=_mesh,
    compiler_params=_sc_params,
    scratch_types=[
        pltpu.VMEM((CH, D), _f32),
        pltpu.VMEM((CH, D), _f32),
        pltpu.VMEM((CH, D), _f32),
        pltpu.VMEM((CH, D), _f32),
        pltpu.VMEM((CH,), jnp.int32),
        pltpu.VMEM((CH,), jnp.int32),
        pltpu.VMEM((CH,), _f32),
        pltpu.VMEM((CH * 17,), _f32),
        pltpu.VMEM((NPAD,), _f32),
        pltpu.VMEM((NPAD,), _f32),
        pltpu.SemaphoreType.DMA,
        pltpu.SemaphoreType.DMA,
    ],
)
def _k3_scores(msg_hbm, recv_hbm, qk_hbm, c_hbm, sc_out, mx_out,
               mra, mrb, qka, qkb, ria, rib, sbuf, pbuf, ccopy, lmax,
               sma, smb):
    wid = _wid()
    bufs = ((mra, qka, ria, sma), (mrb, qkb, rib, smb))
    _fill_vec(lmax, NPAD, -3.0e38)
    pltpu.sync_copy(c_hbm, ccopy)

    pltpu.sync_copy(recv_hbm.at[pl.ds(wid * CH, CH)], ria)
    pltpu.async_copy(msg_hbm.at[pl.ds(wid * CH, CH)], mra, sma)
    pltpu.async_copy(qk_hbm.at[ria], qka, sma)

    iota = lax.broadcasted_iota(jnp.int32, (16,), 0)
    lane0 = iota == 0

    @pl.loop(0, TPW)
    def _(t):
        ch = wid + NW * t
        for ph in range(2):
            m_, q_, r_, s_ = bufs[ph]
            mo_, qo_, ro_, so_ = bufs[1 - ph]

            @pl.when(jnp.logical_and(t % 2 == ph, ch < NCH))
            def _():
                base = ch * CH
                pltpu.make_async_copy(
                    msg_hbm.at[pl.ds(base, CH)], m_, s_).wait()
                pltpu.make_async_copy(qk_hbm.at[r_], q_, s_).wait()
                chn = ch + NW

                @pl.when(chn < NCH)
                def _():
                    nbase = chn * CH
                    pltpu.sync_copy(recv_hbm.at[pl.ds(nbase, CH)], ro_)
                    pltpu.async_copy(msg_hbm.at[pl.ds(nbase, CH)], mo_, so_)
                    pltpu.async_copy(qk_hbm.at[ro_], qo_, so_)

                @pl.loop(0, CH, unroll=4)
                def _(e):
                    acc = jnp.zeros((16,), _f32)
                    for j in range(D // 16):
                        acc = acc + (m_[e, pl.ds(j * 16, 16)]
                                     * q_[e, pl.ds(j * 16, 16)])
                    pbuf[pl.ds(e * 17, 16)] = acc

                for g in range(CH // 16):
                    idx0 = (iota + g * 16) * 17
                    parts = []
                    for c0 in range(4):
                        a = plsc.load_gather(pbuf, [idx0 + c0])
                        for k in range(c0 + 4, 16, 4):
                            a = a + plsc.load_gather(pbuf, [idx0 + k])
                        parts.append(a)
                    sbuf[pl.ds(g * 16, 16)] = (
                        (parts[0] + parts[1]) + (parts[2] + parts[3]))

                for sub in range(CH // 16):
                    r16 = r_[pl.ds(sub * 16, 16)]
                    s16 = (sbuf[pl.ds(sub * 16, 16)]
                           + plsc.load_gather(ccopy, [r16]))
                    sbuf[pl.ds(sub * 16, 16)] = s16
                    _scatter_max(lmax, r16, s16)

                pltpu.sync_copy(sbuf, sc_out.at[pl.ds(base, CH)])

    pltpu.sync_copy(lmax, mx_out.at[pl.ds(wid * NPAD, NPAD)])


@functools.partial(
    pl.kernel,
    out_type=jax.ShapeDtypeStruct((NPAD,), _f32),
    mesh=_mesh,
    compiler_params=_sc_params,
    scratch_types=[
        pltpu.VMEM((NPAD,), _f32),
        pltpu.VMEM((NPAD // NW,), _f32),
    ],
)
def _k4b_merge_max(mx_hbm, m_out, mbuf, obuf):
    wid = _wid()
    span = NPAD // NW
    for p in range(NW):
        pltpu.sync_copy(mx_hbm.at[pl.ds(p * NPAD + wid * span, span)],
                        mbuf.at[pl.ds(p * span, span)])
    for i in range(span // 16):
        acc = mbuf[pl.ds(i * 16, 16)]
        for p in range(1, NW):
            acc = jnp.maximum(acc, mbuf[pl.ds(p * span + i * 16, 16)])
        obuf[pl.ds(i * 16, 16)] = acc
    pltpu.sync_copy(obuf, m_out.at[pl.ds(wid * span, span)])


@functools.partial(
    pl.kernel,
    out_type=jax.ShapeDtypeStruct((NC * NPAD,), _f32),
    mesh=_mesh,
    compiler_params=_sc_params,
    scratch_types=[
        pltpu.VMEM((NPAD,), _f32),
        pltpu.VMEM((CH,), _f32),
        pltpu.VMEM((CH,), _f32),
        pltpu.VMEM((CH,), _f32),
        pltpu.VMEM((CH,), jnp.int32),
        pltpu.VMEM((CH,), jnp.int32),
        pltpu.VMEM((ROWS_PER_TILE,), _f32),
        pltpu.SemaphoreType.DMA,
        pltpu.SemaphoreType.DMA,
        pltpu.VMEM_SHARED((NPAD,), _f32),
    ],
)
def _k4c_z(sc_hbm, recv_hbm, m_hbm, z_out,
           mcopy, sba, sbb, exbuf, ria, rib, zb, sma, smb, spmem_z):
    cid = lax.axis_index("c")
    sid = lax.axis_index("s")
    wid = _wid()
    bufs = ((sba, ria, sma), (sbb, rib, smb))
    _zero_vec(zb, ROWS_PER_TILE)
    pltpu.sync_copy(zb, spmem_z.at[pl.ds(sid * ROWS_PER_TILE, ROWS_PER_TILE)])
    plsc.subcore_barrier()
    pltpu.sync_copy(m_hbm, mcopy)

    pltpu.async_copy(sc_hbm.at[pl.ds(wid * CH, CH)], sba, sma)
    pltpu.async_copy(recv_hbm.at[pl.ds(wid * CH, CH)], ria, sma)

    @pl.loop(0, TPW)
    def _(t):
        ch = wid + NW * t
        for ph in range(2):
            b_, r_, s_ = bufs[ph]
            bo_, ro_, so_ = bufs[1 - ph]

            @pl.when(jnp.logical_and(t % 2 == ph, ch < NCH))
            def _():
                base = ch * CH
                pltpu.make_async_copy(
                    sc_hbm.at[pl.ds(base, CH)], b_, s_).wait()
                pltpu.make_async_copy(
                    recv_hbm.at[pl.ds(base, CH)], r_, s_).wait()
                chn = ch + NW

                @pl.when(chn < NCH)
                def _():
                    nbase = chn * CH
                    pltpu.async_copy(sc_hbm.at[pl.ds(nbase, CH)], bo_, so_)
                    pltpu.async_copy(recv_hbm.at[pl.ds(nbase, CH)], ro_, so_)

                for sub in range(CH // 16):
                    r16 = r_[pl.ds(sub * 16, 16)]
                    m16 = plsc.load_gather(mcopy, [r16])
                    exbuf[pl.ds(sub * 16, 16)] = jnp.exp(
                        b_[pl.ds(sub * 16, 16)] - m16)
                pltpu.sync_copy(exbuf, spmem_z.at[r_], add=True)

    plsc.subcore_barrier()
    pltpu.sync_copy(
        spmem_z.at[pl.ds(sid * ROWS_PER_TILE, ROWS_PER_TILE)],
        z_out.at[pl.ds(cid * NPAD + sid * ROWS_PER_TILE, ROWS_PER_TILE)])


@functools.partial(
    pl.kernel,
    out_type=[
        jax.ShapeDtypeStruct((NPAD,), _f32),
        jax.ShapeDtypeStruct((NPAD,), _f32),
    ],
    mesh=_mesh,
    compiler_params=_sc_params,
    scratch_types=[
        pltpu.VMEM((NC * (NPAD // NW),), _f32),
        pltpu.VMEM((NPAD // NW,), _f32),
        pltpu.VMEM((NPAD // NW,), _f32),
        pltpu.VMEM((NPAD // NW,), _f32),
    ],
)
def _k4d_r_t(z_hbm, m_hbm, v_out, t_out, zbuf, mbuf, rbuf, tbuf):
    wid = _wid()
    span = NPAD // NW
    for p in range(NC):
        pltpu.sync_copy(z_hbm.at[pl.ds(p * NPAD + wid * span, span)],
                        zbuf.at[pl.ds(p * span, span)])
    pltpu.sync_copy(m_hbm.at[pl.ds(wid * span, span)], mbuf)
    for i in range(span // 16):
        z = zbuf[pl.ds(i * 16, 16)] + zbuf[pl.ds(span + i * 16, 16)]
        r = 1.0 / (z + 1e-8)
        rbuf[pl.ds(i * 16, 16)] = r * jnp.exp(-mbuf[pl.ds(i * 16, 16)])
        tbuf[pl.ds(i * 16, 16)] = z * r
    pltpu.sync_copy(rbuf, v_out.at[pl.ds(wid * span, span)])
    pltpu.sync_copy(tbuf, t_out.at[pl.ds(wid * span, span)])


@functools.partial(
    pl.kernel,
    out_type=jax.ShapeDtypeStruct((NC, NPAD, D), _f32),
    mesh=_mesh,
    compiler_params=_sc_params,
    scratch_types=[
        pltpu.VMEM((CH, D), _f32),
        pltpu.VMEM((CH, D), _f32),
        pltpu.VMEM((CH,), jnp.int32),
        pltpu.VMEM((CH,), jnp.int32),
        pltpu.VMEM((CH,), _f32),
        pltpu.VMEM((CH,), _f32),
        pltpu.VMEM((CH,), _f32),
        pltpu.VMEM((NPAD,), _f32),
        pltpu.SemaphoreType.DMA,
        pltpu.SemaphoreType.DMA,
        pltpu.VMEM_SHARED((NPAD, D), _f32),
    ],
)
def _k5_weighted_sum(msg_hbm, recv_hbm, sc_hbm, v_hbm, a_out,
                     mra, mrb, ria, rib, sca, scb, wbuf, vcopy,
                     sma, smb, spmem):
    cid = lax.axis_index("c")
    sid = lax.axis_index("s")
    wid = _wid()
    bufs = ((mra, ria, sca, sma), (mrb, rib, scb, smb))
    _zero_rows(mra, CH)
    for t in range(ROWS_PER_TILE // CH):
        pltpu.sync_copy(mra, spmem.at[pl.ds(sid * ROWS_PER_TILE + t * CH, CH)])
    plsc.subcore_barrier()
    pltpu.sync_copy(v_hbm, vcopy)

    pltpu.async_copy(recv_hbm.at[pl.ds(wid * CH, CH)], ria, sma)
    pltpu.async_copy(sc_hbm.at[pl.ds(wid * CH, CH)], sca, sma)
    pltpu.async_copy(msg_hbm.at[pl.ds(wid * CH, CH)], mra, sma)

    @pl.loop(0, TPW)
    def _(t):
        ch = wid + NW * t
        for ph in range(2):
            m_, r_, c_, s_ = bufs[ph]
            mo_, ro_, co_, so_ = bufs[1 - ph]

            @pl.when(jnp.logical_and(t % 2 == ph, ch < NCH))
            def _():
                base = ch * CH
                pltpu.make_async_copy(
                    recv_hbm.at[pl.ds(base, CH)], r_, s_).wait()
                pltpu.make_async_copy(
                    sc_hbm.at[pl.ds(base, CH)], c_, s_).wait()
                pltpu.make_async_copy(
                    msg_hbm.at[pl.ds(base, CH)], m_, s_).wait()
                chn = ch + NW

                @pl.when(chn < NCH)
                def _():
                    nbase = chn * CH
                    pltpu.async_copy(recv_hbm.at[pl.ds(nbase, CH)], ro_, so_)
                    pltpu.async_copy(sc_hbm.at[pl.ds(nbase, CH)], co_, so_)
                    pltpu.async_copy(msg_hbm.at[pl.ds(nbase, CH)], mo_, so_)

                for sub in range(CH // 16):
                    r16 = r_[pl.ds(sub * 16, 16)]
                    v16 = plsc.load_gather(vcopy, [r16])
                    ex16 = jnp.exp(c_[pl.ds(sub * 16, 16)])
                    wbuf[pl.ds(sub * 16, 16)] = ex16 * v16

                @pl.loop(0, CH, unroll=4)
                def _(e):
                    wsplat = plsc.load_gather(
                        wbuf, [jnp.full((16,), e, jnp.int32)])
                    for j in range(D // 16):
                        m_[e, pl.ds(j * 16, 16)] = (
                            m_[e, pl.ds(j * 16, 16)] * wsplat)

                pltpu.sync_copy(m_, spmem.at[r_], add=True)

    plsc.subcore_barrier()
    pltpu.sync_copy(
        spmem.at[pl.ds(sid * ROWS_PER_TILE, ROWS_PER_TILE)],
        a_out.at[cid, pl.ds(sid * ROWS_PER_TILE, ROWS_PER_TILE)])


def _k6_body(a0, a1, t, wv, bv, wo, bo, out):
    a = a0[...] + a1[...]
    agg = (jnp.dot(a, wv[...], preferred_element_type=_f32)
           + t[...][:, None] * bv[...][None, :])
    out[...] = jnp.dot(agg, wo[...], preferred_element_type=_f32) + bo[...][None, :]


def _k6_project(a0, a1, t, wv, bv, wo, bo):
    nblk = NPAD // 512
    return pl.pallas_call(
        _k6_body,
        grid=(nblk,),
        in_specs=[
            pl.BlockSpec((512, D), lambda i: (i, 0)),
            pl.BlockSpec((512, D), lambda i: (i, 0)),
            pl.BlockSpec((512,), lambda i: (i,)),
            pl.BlockSpec((D, D), lambda i: (0, 0)),
            pl.BlockSpec((D,), lambda i: (0,)),
            pl.BlockSpec((D, D), lambda i: (0, 0)),
            pl.BlockSpec((D,), lambda i: (0,)),
        ],
        out_specs=pl.BlockSpec((512, D), lambda i: (i, 0)),
        out_shape=jax.ShapeDtypeStruct((NPAD, D), _f32),
    )(a0, a1, t, wv, bv, wo, bo)


def kernel(messages, receivers, num_segments, Wk, bk, Wv, bv, Wq, bq, Wo, bo):
    receivers = receivers.astype(jnp.int32)
    s_parts = _k1_segment_sum(messages, receivers)
    qk, c = _k2_tables(s_parts[0], s_parts[1], Wq, bq, Wk, bk)
    scores, mx = _k3_scores(messages, receivers, qk, c)
    m = _k4b_merge_max(mx)
    z = _k4c_z(scores, receivers, m)
    v, t = _k4d_r_t(z, m)
    a_parts = _k5_weighted_sum(messages, receivers, scores, v)
    out = _k6_project(a_parts[0], a_parts[1], t, Wv, bv, Wo, bo)
    return out[:N]


# speedup vs baseline: 1.0919x; 1.0067x over previous
import functools

import jax
import jax.numpy as jnp
from jax import lax
from jax.experimental import pallas as pl
from jax.experimental.pallas import tpu as pltpu
from jax.experimental.pallas import tpu_sc as plsc

E = 320000
N = 10000
D = 128
NPAD = 10240
NC = 2
NS = 16
NW = NC * NS
CH = 128
NCH = E // CH
TPW = (NCH + NW - 1) // NW
ROWS_PER_TILE = NPAD // NS
INV_SQRT_D = 1.0 / (D ** 0.5)

_mesh = plsc.VectorSubcoreMesh(
    core_axis_name="c", subcore_axis_name="s", num_cores=NC, num_subcores=NS)

_sc_params = pltpu.CompilerParams(needs_layout_passes=False)

_f32 = jnp.float32


def _wid():
    return lax.axis_index("s") * NC + lax.axis_index("c")


def _zero_rows(buf, n_rows):
    zero16 = jnp.zeros((16,), _f32)

    @pl.loop(0, n_rows)
    def _(i):
        for j in range(D // 16):
            buf[i, pl.ds(j * 16, 16)] = zero16


def _zero_vec(buf, n):
    zero16 = jnp.zeros((16,), _f32)

    @pl.loop(0, n // 16)
    def _(i):
        buf[pl.ds(i * 16, 16)] = zero16


def _fill_vec(buf, n, val):
    v16 = jnp.full((16,), val, _f32)

    @pl.loop(0, n // 16)
    def _(i):
        buf[pl.ds(i * 16, 16)] = v16


def _scatter_max(lmax, r16, v16):
    def cond(m):
        return plsc.all_reduce_population_count(m)[0] > 0

    def body(m):
        cur = plsc.load_gather(lmax, [r16])
        need = jnp.logical_and(m, cur < v16)
        plsc.store_scatter(lmax, [r16], v16, mask=need)
        return need

    lax.while_loop(cond, body, jnp.ones((16,), jnp.bool_))


@functools.partial(
    pl.kernel,
    out_type=jax.ShapeDtypeStruct((NC, NPAD, D), _f32),
    mesh=_mesh,
    compiler_params=_sc_params,
    scratch_types=[
        pltpu.VMEM((CH, D), _f32),
        pltpu.VMEM((CH, D), _f32),
        pltpu.VMEM((CH,), jnp.int32),
        pltpu.VMEM((CH,), jnp.int32),
        pltpu.SemaphoreType.DMA,
        pltpu.SemaphoreType.DMA,
        pltpu.SemaphoreType.DMA,
        pltpu.SemaphoreType.DMA,
        pltpu.VMEM_SHARED((NPAD, D), _f32),
    ],
)
def _k1_segment_sum(msg_hbm, recv_hbm, s_out, mra, mrb, ria, rib, sma, smb,
                    ssa, ssb, spmem):
    cid = lax.axis_index("c")
    sid = lax.axis_index("s")
    wid = _wid()
    bufs = ((mra, ria, sma, ssa), (mrb, rib, smb, ssb))
    _zero_rows(mra, CH)
    for t in range(ROWS_PER_TILE // CH):
        pltpu.sync_copy(mra, spmem.at[pl.ds(sid * ROWS_PER_TILE + t * CH, CH)])
    plsc.subcore_barrier()

    pltpu.async_copy(recv_hbm.at[pl.ds(wid * CH, CH)], ria, sma)
    pltpu.async_copy(msg_hbm.at[pl.ds(wid * CH, CH)], mra, sma)

    @pl.loop(0, TPW)
    def _(t):
        ch = wid + NW * t
        for ph in range(2):
            m_, r_, s_, ss_ = bufs[ph]
            mo_, ro_, so_, sso_ = bufs[1 - ph]

            @pl.when(jnp.logical_and(t % 2 == ph, ch < NCH))
            def _():
                base = ch * CH
                pltpu.make_async_copy(
                    recv_hbm.at[pl.ds(base, CH)], r_, s_).wait()
                pltpu.make_async_copy(
                    msg_hbm.at[pl.ds(base, CH)], m_, s_).wait()
                chn = ch + NW

                @pl.when(chn < NCH)
                def _():
                    @pl.when(t > 0)
                    def _():
                        pltpu.make_async_copy(
                            mo_, spmem.at[ro_], sso_).wait()

                    nbase = chn * CH
                    pltpu.async_copy(recv_hbm.at[pl.ds(nbase, CH)], ro_, so_)
                    pltpu.async_copy(msg_hbm.at[pl.ds(nbase, CH)], mo_, so_)

                pltpu.async_copy(m_, spmem.at[r_], ss_, add=True)

    pltpu.make_async_copy(mra, spmem.at[ria], ssa).wait()
    pltpu.make_async_copy(mrb, spmem.at[rib], ssb).wait()
    plsc.subcore_barrier()
    pltpu.sync_copy(
        spmem.at[pl.ds(sid * ROWS_PER_TILE, ROWS_PER_TILE)],
        s_out.at[cid, pl.ds(sid * ROWS_PER_TILE, ROWS_PER_TILE)])


def _k2_body(s0, s1, wq, bq, wk, bk, qk_out, c_out):
    s = s0[...] + s1[...]
    q = jnp.dot(s, wq[...], preferred_element_type=_f32) + bq[...][None, :]
    qk = lax.dot_general(q, wk[...], (((1,), (1,)), ((), ())),
                         preferred_element_type=_f32)
    qk_out[...] = qk * INV_SQRT_D
    c = jnp.dot(q, bk[...], preferred_element_type=_f32)
    c_out[...] = c * INV_SQRT_D


def _k2_tables(s0, s1, wq, bq, wk, bk):
    nblk = NPAD // 512
    return pl.pallas_call(
        _k2_body,
        grid=(nblk,),
        in_specs=[
            pl.BlockSpec((512, D), lambda i: (i, 0)),
            pl.BlockSpec((512, D), lambda i: (i, 0)),
            pl.BlockSpec((D, D), lambda i: (0, 0)),
            pl.BlockSpec((D,), lambda i: (0,)),
            pl.BlockSpec((D, D), lambda i: (0, 0)),
            pl.BlockSpec((D,), lambda i: (0,)),
        ],
        out_specs=[
            pl.BlockSpec((512, D), lambda i: (i, 0)),
            pl.BlockSpec((512,), lambda i: (i,)),
        ],
        out_shape=[
            jax.ShapeDtypeStruct((NPAD, D), _f32),
            jax.ShapeDtypeStruct((NPAD,), _f32),
        ],
    )(s0, s1, wq, bq, wk, bk)


@functools.partial(
    pl.kernel,
    out_type=[
        jax.ShapeDtypeStruct((E,), _f32),
        jax.ShapeDtypeStruct((NW * NPAD,), _f32),
    ],
    mesh=_mesh,
    compiler_params=_sc_params,
    scratch_types=[
        pltpu.VMEM((CH, D), _f32),
        pltpu.VMEM((CH, D), _f32),
        pltpu.VMEM((CH, D), _f32),
        pltpu.VMEM((CH, D), _f32),
        pltpu.VMEM((CH,), jnp.int32),
        pltpu.VMEM((CH,), jnp.int32),
        pltpu.VMEM((CH,), _f32),
        pltpu.VMEM((CH * 17,), _f32),
        pltpu.VMEM((NPAD,), _f32),
        pltpu.VMEM((NPAD,), _f32),
        pltpu.SemaphoreType.DMA,
        pltpu.SemaphoreType.DMA,
    ],
)
def _k3_scores(msg_hbm, recv_hbm, qk_hbm, c_hbm, sc_out, mx_out,
               mra, mrb, qka, qkb, ria, rib, sbuf, pbuf, ccopy, lmax,
               sma, smb):
    wid = _wid()
    bufs = ((mra, qka, ria, sma), (mrb, qkb, rib, smb))
    _fill_vec(lmax, NPAD, -3.0e38)
    pltpu.sync_copy(c_hbm, ccopy)

    pltpu.sync_copy(recv_hbm.at[pl.ds(wid * CH, CH)], ria)
    pltpu.async_copy(msg_hbm.at[pl.ds(wid * CH, CH)], mra, sma)
    pltpu.async_copy(qk_hbm.at[ria], qka, sma)

    iota = lax.broadcasted_iota(jnp.int32, (16,), 0)
    lane0 = iota == 0

    @pl.loop(0, TPW)
    def _(t):
        ch = wid + NW * t
        for ph in range(2):
            m_, q_, r_, s_ = bufs[ph]
            mo_, qo_, ro_, so_ = bufs[1 - ph]

            @pl.when(jnp.logical_and(t % 2 == ph, ch < NCH))
            def _():
                base = ch * CH
                chn = ch + NW

                @pl.when(chn < NCH)
                def _():
                    pltpu.async_copy(
                        recv_hbm.at[pl.ds(chn * CH, CH)], ro_, so_)

                pltpu.make_async_copy(
                    msg_hbm.at[pl.ds(base, CH)], m_, s_).wait()
                pltpu.make_async_copy(qk_hbm.at[r_], q_, s_).wait()

                @pl.when(chn < NCH)
                def _():
                    nbase = chn * CH
                    pltpu.make_async_copy(
                        recv_hbm.at[pl.ds(nbase, CH)], ro_, so_).wait()
                    pltpu.async_copy(msg_hbm.at[pl.ds(nbase, CH)], mo_, so_)
                    pltpu.async_copy(qk_hbm.at[ro_], qo_, so_)

                @pl.loop(0, CH, unroll=4)
                def _(e):
                    acc = jnp.zeros((16,), _f32)
                    for j in range(D // 16):
                        acc = acc + (m_[e, pl.ds(j * 16, 16)]
                                     * q_[e, pl.ds(j * 16, 16)])
                    pbuf[pl.ds(e * 17, 16)] = acc

                for g in range(CH // 16):
                    idx0 = (iota + g * 16) * 17
                    parts = []
                    for c0 in range(4):
                        a = plsc.load_gather(pbuf, [idx0 + c0])
                        for k in range(c0 + 4, 16, 4):
                            a = a + plsc.load_gather(pbuf, [idx0 + k])
                        parts.append(a)
                    sbuf[pl.ds(g * 16, 16)] = (
                        (parts[0] + parts[1]) + (parts[2] + parts[3]))

                for sub in range(CH // 16):
                    r16 = r_[pl.ds(sub * 16, 16)]
                    s16 = (sbuf[pl.ds(sub * 16, 16)]
                           + plsc.load_gather(ccopy, [r16]))
                    sbuf[pl.ds(sub * 16, 16)] = s16
                    _scatter_max(lmax, r16, s16)

                pltpu.sync_copy(sbuf, sc_out.at[pl.ds(base, CH)])

    pltpu.sync_copy(lmax, mx_out.at[pl.ds(wid * NPAD, NPAD)])


@functools.partial(
    pl.kernel,
    out_type=jax.ShapeDtypeStruct((NPAD,), _f32),
    mesh=_mesh,
    compiler_params=_sc_params,
    scratch_types=[
        pltpu.VMEM((NPAD,), _f32),
        pltpu.VMEM((NPAD // NW,), _f32),
    ],
)
def _k4b_merge_max(mx_hbm, m_out, mbuf, obuf):
    wid = _wid()
    span = NPAD // NW
    for p in range(NW):
        pltpu.sync_copy(mx_hbm.at[pl.ds(p * NPAD + wid * span, span)],
                        mbuf.at[pl.ds(p * span, span)])
    for i in range(span // 16):
        acc = mbuf[pl.ds(i * 16, 16)]
        for p in range(1, NW):
            acc = jnp.maximum(acc, mbuf[pl.ds(p * span + i * 16, 16)])
        obuf[pl.ds(i * 16, 16)] = acc
    pltpu.sync_copy(obuf, m_out.at[pl.ds(wid * span, span)])


@functools.partial(
    pl.kernel,
    out_type=jax.ShapeDtypeStruct((NC * NPAD,), _f32),
    mesh=_mesh,
    compiler_params=_sc_params,
    scratch_types=[
        pltpu.VMEM((NPAD,), _f32),
        pltpu.VMEM((CH,), _f32),
        pltpu.VMEM((CH,), _f32),
        pltpu.VMEM((CH,), _f32),
        pltpu.VMEM((CH,), jnp.int32),
        pltpu.VMEM((CH,), jnp.int32),
        pltpu.VMEM((ROWS_PER_TILE,), _f32),
        pltpu.SemaphoreType.DMA,
        pltpu.SemaphoreType.DMA,
        pltpu.VMEM_SHARED((NPAD,), _f32),
    ],
)
def _k4c_z(sc_hbm, recv_hbm, m_hbm, z_out,
           mcopy, sba, sbb, exbuf, ria, rib, zb, sma, smb, spmem_z):
    cid = lax.axis_index("c")
    sid = lax.axis_index("s")
    wid = _wid()
    bufs = ((sba, ria, sma), (sbb, rib, smb))
    _zero_vec(zb, ROWS_PER_TILE)
    pltpu.sync_copy(zb, spmem_z.at[pl.ds(sid * ROWS_PER_TILE, ROWS_PER_TILE)])
    plsc.subcore_barrier()
    pltpu.sync_copy(m_hbm, mcopy)

    pltpu.async_copy(sc_hbm.at[pl.ds(wid * CH, CH)], sba, sma)
    pltpu.async_copy(recv_hbm.at[pl.ds(wid * CH, CH)], ria, sma)

    @pl.loop(0, TPW)
    def _(t):
        ch = wid + NW * t
        for ph in range(2):
            b_, r_, s_ = bufs[ph]
            bo_, ro_, so_ = bufs[1 - ph]

            @pl.when(jnp.logical_and(t % 2 == ph, ch < NCH))
            def _():
                base = ch * CH
                pltpu.make_async_copy(
                    sc_hbm.at[pl.ds(base, CH)], b_, s_).wait()
                pltpu.make_async_copy(
                    recv_hbm.at[pl.ds(base, CH)], r_, s_).wait()
                chn = ch + NW

                @pl.when(chn < NCH)
                def _():
                    nbase = chn * CH
                    pltpu.async_copy(sc_hbm.at[pl.ds(nbase, CH)], bo_, so_)
                    pltpu.async_copy(recv_hbm.at[pl.ds(nbase, CH)], ro_, so_)

                for sub in range(CH // 16):
                    r16 = r_[pl.ds(sub * 16, 16)]
                    m16 = plsc.load_gather(mcopy, [r16])
                    exbuf[pl.ds(sub * 16, 16)] = jnp.exp(
                        b_[pl.ds(sub * 16, 16)] - m16)
                pltpu.sync_copy(exbuf, spmem_z.at[r_], add=True)

    plsc.subcore_barrier()
    pltpu.sync_copy(
        spmem_z.at[pl.ds(sid * ROWS_PER_TILE, ROWS_PER_TILE)],
        z_out.at[pl.ds(cid * NPAD + sid * ROWS_PER_TILE, ROWS_PER_TILE)])


@functools.partial(
    pl.kernel,
    out_type=[
        jax.ShapeDtypeStruct((NPAD,), _f32),
        jax.ShapeDtypeStruct((NPAD,), _f32),
    ],
    mesh=_mesh,
    compiler_params=_sc_params,
    scratch_types=[
        pltpu.VMEM((NC * (NPAD // NW),), _f32),
        pltpu.VMEM((NPAD // NW,), _f32),
        pltpu.VMEM((NPAD // NW,), _f32),
        pltpu.VMEM((NPAD // NW,), _f32),
    ],
)
def _k4d_r_t(z_hbm, m_hbm, v_out, t_out, zbuf, mbuf, rbuf, tbuf):
    wid = _wid()
    span = NPAD // NW
    for p in range(NC):
        pltpu.sync_copy(z_hbm.at[pl.ds(p * NPAD + wid * span, span)],
                        zbuf.at[pl.ds(p * span, span)])
    pltpu.sync_copy(m_hbm.at[pl.ds(wid * span, span)], mbuf)
    for i in range(span // 16):
        z = zbuf[pl.ds(i * 16, 16)] + zbuf[pl.ds(span + i * 16, 16)]
        r = 1.0 / (z + 1e-8)
        rbuf[pl.ds(i * 16, 16)] = r * jnp.exp(-mbuf[pl.ds(i * 16, 16)])
        tbuf[pl.ds(i * 16, 16)] = z * r
    pltpu.sync_copy(rbuf, v_out.at[pl.ds(wid * span, span)])
    pltpu.sync_copy(tbuf, t_out.at[pl.ds(wid * span, span)])


@functools.partial(
    pl.kernel,
    out_type=jax.ShapeDtypeStruct((NC, NPAD, D), _f32),
    mesh=_mesh,
    compiler_params=_sc_params,
    scratch_types=[
        pltpu.VMEM((CH, D), _f32),
        pltpu.VMEM((CH, D), _f32),
        pltpu.VMEM((CH,), jnp.int32),
        pltpu.VMEM((CH,), jnp.int32),
        pltpu.VMEM((CH,), _f32),
        pltpu.VMEM((CH,), _f32),
        pltpu.VMEM((CH,), _f32),
        pltpu.VMEM((NPAD,), _f32),
        pltpu.SemaphoreType.DMA,
        pltpu.SemaphoreType.DMA,
        pltpu.SemaphoreType.DMA,
        pltpu.SemaphoreType.DMA,
        pltpu.VMEM_SHARED((NPAD, D), _f32),
    ],
)
def _k5_weighted_sum(msg_hbm, recv_hbm, sc_hbm, v_hbm, a_out,
                     mra, mrb, ria, rib, sca, scb, wbuf, vcopy,
                     sma, smb, ssa, ssb, spmem):
    cid = lax.axis_index("c")
    sid = lax.axis_index("s")
    wid = _wid()
    bufs = ((mra, ria, sca, sma, ssa), (mrb, rib, scb, smb, ssb))
    _zero_rows(mra, CH)
    for t in range(ROWS_PER_TILE // CH):
        pltpu.sync_copy(mra, spmem.at[pl.ds(sid * ROWS_PER_TILE + t * CH, CH)])
    plsc.subcore_barrier()
    pltpu.sync_copy(v_hbm, vcopy)

    pltpu.async_copy(recv_hbm.at[pl.ds(wid * CH, CH)], ria, sma)
    pltpu.async_copy(sc_hbm.at[pl.ds(wid * CH, CH)], sca, sma)
    pltpu.async_copy(msg_hbm.at[pl.ds(wid * CH, CH)], mra, sma)

    @pl.loop(0, TPW)
    def _(t):
        ch = wid + NW * t
        for ph in range(2):
            m_, r_, c_, s_, ss_ = bufs[ph]
            mo_, ro_, co_, so_, sso_ = bufs[1 - ph]

            @pl.when(jnp.logical_and(t % 2 == ph, ch < NCH))
            def _():
                base = ch * CH
                pltpu.make_async_copy(
                    recv_hbm.at[pl.ds(base, CH)], r_, s_).wait()
                pltpu.make_async_copy(
                    sc_hbm.at[pl.ds(base, CH)], c_, s_).wait()
                pltpu.make_async_copy(
                    msg_hbm.at[pl.ds(base, CH)], m_, s_).wait()
                chn = ch + NW

                @pl.when(chn < NCH)
                def _():
                    @pl.when(t > 0)
                    def _():
                        pltpu.make_async_copy(
                            mo_, spmem.at[ro_], sso_).wait()

                    nbase = chn * CH
                    pltpu.async_copy(recv_hbm.at[pl.ds(nbase, CH)], ro_, so_)
                    pltpu.async_copy(sc_hbm.at[pl.ds(nbase, CH)], co_, so_)
                    pltpu.async_copy(msg_hbm.at[pl.ds(nbase, CH)], mo_, so_)

                for sub in range(CH // 16):
                    r16 = r_[pl.ds(sub * 16, 16)]
                    v16 = plsc.load_gather(vcopy, [r16])
                    ex16 = jnp.exp(c_[pl.ds(sub * 16, 16)])
                    wbuf[pl.ds(sub * 16, 16)] = ex16 * v16

                @pl.loop(0, CH, unroll=4)
                def _(e):
                    wsplat = plsc.load_gather(
                        wbuf, [jnp.full((16,), e, jnp.int32)])
                    for j in range(D // 16):
                        m_[e, pl.ds(j * 16, 16)] = (
                            m_[e, pl.ds(j * 16, 16)] * wsplat)

                pltpu.async_copy(m_, spmem.at[r_], ss_, add=True)

    pltpu.make_async_copy(mra, spmem.at[ria], ssa).wait()
    pltpu.make_async_copy(mrb, spmem.at[rib], ssb).wait()
    plsc.subcore_barrier()
    pltpu.sync_copy(
        spmem.at[pl.ds(sid * ROWS_PER_TILE, ROWS_PER_TILE)],
        a_out.at[cid, pl.ds(sid * ROWS_PER_TILE, ROWS_PER_TILE)])


def _k6_body(a0, a1, t, wv, bv, wo, bo, out):
    a = a0[...] + a1[...]
    agg = (jnp.dot(a, wv[...], preferred_element_type=_f32)
           + t[...][:, None] * bv[...][None, :])
    out[...] = jnp.dot(agg, wo[...], preferred_element_type=_f32) + bo[...][None, :]


def _k6_project(a0, a1, t, wv, bv, wo, bo):
    nblk = NPAD // 512
    return pl.pallas_call(
        _k6_body,
        grid=(nblk,),
        in_specs=[
            pl.BlockSpec((512, D), lambda i: (i, 0)),
            pl.BlockSpec((512, D), lambda i: (i, 0)),
            pl.BlockSpec((512,), lambda i: (i,)),
            pl.BlockSpec((D, D), lambda i: (0, 0)),
            pl.BlockSpec((D,), lambda i: (0,)),
            pl.BlockSpec((D, D), lambda i: (0, 0)),
            pl.BlockSpec((D,), lambda i: (0,)),
        ],
        out_specs=pl.BlockSpec((512, D), lambda i: (i, 0)),
        out_shape=jax.ShapeDtypeStruct((NPAD, D), _f32),
    )(a0, a1, t, wv, bv, wo, bo)


def kernel(messages, receivers, num_segments, Wk, bk, Wv, bv, Wq, bq, Wo, bo):
    receivers = receivers.astype(jnp.int32)
    s_parts = _k1_segment_sum(messages, receivers)
    qk, c = _k2_tables(s_parts[0], s_parts[1], Wq, bq, Wk, bk)
    scores, mx = _k3_scores(messages, receivers, qk, c)
    m = _k4b_merge_max(mx)
    z = _k4c_z(scores, receivers, m)
    v, t = _k4d_r_t(z, m)
    a_parts = _k5_weighted_sum(messages, receivers, scores, v)
    out = _k6_project(a_parts[0], a_parts[1], t, Wv, bv, Wo, bo)
    return out[:N]
